# Initial kernel scaffold; baseline (speedup 1.0000x reference)
import jax
import jax.numpy as jnp
from jax.experimental import pallas as pl


def kernel(outputs, complementary_labels):
    raise NotImplementedError("write your pallas kernel here")


# speedup vs baseline: 6.2419x; 6.2419x over previous
import functools

import jax
import jax.numpy as jnp
from jax import lax
from jax.experimental import pallas as pl
from jax.experimental.pallas import tpu as pltpu

_NCLS = 1000
_ROWS = 256


def _block_body(x_ref, lab_ref, acc_ref):
    x = x_ref[...]
    labs = lab_ref[...]
    m = jnp.max(x, axis=1, keepdims=True)
    e = jnp.exp(x - m)
    z = jnp.sum(e, axis=1)
    col = lax.broadcasted_iota(jnp.int32, x.shape, 1)
    mask = col == labs[:, 0:1]
    for j in range(1, labs.shape[1]):
        mask = jnp.logical_or(mask, col == labs[:, j : j + 1])
    sum_in = jnp.sum(jnp.where(mask, e, 0.0), axis=1)
    frac = jnp.maximum(z - sum_in, 0.0) / z
    loss = -jnp.log(frac + 1e-7)
    ncomp = jnp.sum((labs != -1).astype(jnp.float32), axis=1)
    scale = (_NCLS - 1.0) / (_NCLS - ncomp)
    acc_ref[...] = jnp.sum(scale * loss)[None, None, None]


@jax.jit
def kernel(outputs, complementary_labels):
    batch, ncls = outputs.shape
    labs = complementary_labels.astype(jnp.int32)
    nblocks = batch // _ROWS
    partials = pl.pallas_call(
        _block_body,
        grid=(nblocks,),
        in_specs=[
            pl.BlockSpec((_ROWS, ncls), lambda i: (i, 0)),
            pl.BlockSpec((_ROWS, labs.shape[1]), lambda i: (i, 0)),
        ],
        out_specs=pl.BlockSpec((1, 1, 1), lambda i: (i, 0, 0)),
        out_shape=jax.ShapeDtypeStruct((nblocks, 1, 1), jnp.float32),
        compiler_params=pltpu.CompilerParams(
            dimension_semantics=("parallel",),
        ),
    )(outputs, labs)
    return jnp.sum(partials) / batch
